# baseline (speedup 1.0000x reference)
import functools

import jax
import jax.numpy as jnp
from jax.experimental import pallas as pl
from jax.experimental.pallas import tpu as pltpu


def _posenc_kernel(x_ref, row_ref, col_ref, o_ref, pos_ref, *, H, W, CH, BS, NJ):
    i = pl.program_id(0)
    HW = H * W

    @pl.when(i == 0)
    def _build_pos():
        r = jax.lax.broadcasted_iota(jnp.int32, (HW, H), 0)
        k = jax.lax.broadcasted_iota(jnp.int32, (HW, H), 1)
        Eh = (r // W == k).astype(jnp.float32)
        Ew = (r % W == k).astype(jnp.float32)
        pos_ref[:, :CH] = jax.lax.dot(
            Eh, row_ref[:H, :], precision=jax.lax.Precision.HIGHEST,
            preferred_element_type=jnp.float32)
        pos_ref[:, CH:] = jax.lax.dot(
            Ew, col_ref[:W, :], precision=jax.lax.Precision.HIGHEST,
            preferred_element_type=jnp.float32)

    off = (i % NJ) * BS
    o_ref[...] = x_ref[...] + pos_ref[pl.ds(off, BS), :]


def kernel(x, row_embed, col_embed):
    b, c, h, w = x.shape
    ch = c // 2
    hw = h * w
    BS = 192
    nj = hw // BS
    xt = jnp.transpose(x, (0, 2, 3, 1)).reshape(b * hw, c)
    body = functools.partial(_posenc_kernel, H=h, W=w, CH=ch, BS=BS, NJ=nj)
    out = pl.pallas_call(
        body,
        grid=(b * nj,),
        in_specs=[
            pl.BlockSpec((BS, c), lambda i: (i, 0)),
            pl.BlockSpec(row_embed.shape, lambda i: (0, 0)),
            pl.BlockSpec(col_embed.shape, lambda i: (0, 0)),
        ],
        out_specs=pl.BlockSpec((BS, c), lambda i: (i, 0)),
        out_shape=jax.ShapeDtypeStruct((b * hw, c), x.dtype),
        scratch_shapes=[pltpu.VMEM((hw, c), jnp.float32)],
    )(xt, row_embed, col_embed)
    return out.reshape(b, h, w, c).transpose(0, 3, 1, 2)


# speedup vs baseline: 2.6617x; 2.6617x over previous
import functools

import jax
import jax.numpy as jnp
from jax.experimental import pallas as pl
from jax.experimental.pallas import tpu as pltpu


def _posenc_kernel(x_ref, row_ref, col_ref, o_ref, pos_ref, *, H, W, CH):
    i = pl.program_id(0)
    HW = H * W

    @pl.when(i == 0)
    def _build_pos():
        r = jax.lax.broadcasted_iota(jnp.int32, (HW, H), 0)
        k = jax.lax.broadcasted_iota(jnp.int32, (HW, H), 1)
        Eh = (r // W == k).astype(jnp.float32)
        Ew = (r % W == k).astype(jnp.float32)
        pos_ref[:, :CH] = jax.lax.dot(
            Eh, row_ref[:H, :], precision=jax.lax.Precision.HIGHEST,
            preferred_element_type=jnp.float32)
        pos_ref[:, CH:] = jax.lax.dot(
            Ew, col_ref[:W, :], precision=jax.lax.Precision.HIGHEST,
            preferred_element_type=jnp.float32)

    o_ref[...] = x_ref[...] + pos_ref[...][None]


def kernel(x, row_embed, col_embed):
    b, c, h, w = x.shape
    ch = c // 2
    hw = h * w
    G = 2
    xt = jnp.transpose(x, (0, 2, 3, 1)).reshape(b, hw, c)
    body = functools.partial(_posenc_kernel, H=h, W=w, CH=ch)
    out = pl.pallas_call(
        body,
        grid=(b // G,),
        in_specs=[
            pl.BlockSpec((G, hw, c), lambda i: (i, 0, 0)),
            pl.BlockSpec(row_embed.shape, lambda i: (0, 0)),
            pl.BlockSpec(col_embed.shape, lambda i: (0, 0)),
        ],
        out_specs=pl.BlockSpec((G, hw, c), lambda i: (i, 0, 0)),
        out_shape=jax.ShapeDtypeStruct((b, hw, c), x.dtype),
        scratch_shapes=[pltpu.VMEM((hw, c), jnp.float32)],
    )(xt, row_embed, col_embed)
    return out.reshape(b, h, w, c).transpose(0, 3, 1, 2)


# speedup vs baseline: 2.9731x; 1.1170x over previous
import functools

import jax
import jax.numpy as jnp
from jax.experimental import pallas as pl
from jax.experimental.pallas import tpu as pltpu


def _posenc_kernel(x_ref, row_ref, col_ref, o_ref, pos_ref, *, H, W, CH):
    i = pl.program_id(0)
    HW = H * W

    @pl.when(i == 0)
    def _build_pos():
        r = jax.lax.broadcasted_iota(jnp.int32, (HW, H), 0)
        k = jax.lax.broadcasted_iota(jnp.int32, (HW, H), 1)
        Eh = (r // W == k).astype(jnp.float32)
        Ew = (r % W == k).astype(jnp.float32)
        pos_ref[:, :CH] = jax.lax.dot(
            Eh, row_ref[:H, :], precision=jax.lax.Precision.HIGHEST,
            preferred_element_type=jnp.float32)
        pos_ref[:, CH:] = jax.lax.dot(
            Ew, col_ref[:W, :], precision=jax.lax.Precision.HIGHEST,
            preferred_element_type=jnp.float32)

    o_ref[...] = x_ref[...] + pos_ref[...][None]


def kernel(x, row_embed, col_embed):
    b, c, h, w = x.shape
    ch = c // 2
    hw = h * w
    G = 4
    xt = jnp.transpose(x, (0, 2, 3, 1)).reshape(b, hw, c)
    body = functools.partial(_posenc_kernel, H=h, W=w, CH=ch)
    out = pl.pallas_call(
        body,
        grid=(b // G,),
        in_specs=[
            pl.BlockSpec((G, hw, c), lambda i: (i, 0, 0)),
            pl.BlockSpec(row_embed.shape, lambda i: (0, 0)),
            pl.BlockSpec(col_embed.shape, lambda i: (0, 0)),
        ],
        out_specs=pl.BlockSpec((G, hw, c), lambda i: (i, 0, 0)),
        out_shape=jax.ShapeDtypeStruct((b, hw, c), x.dtype),
        scratch_shapes=[pltpu.VMEM((hw, c), jnp.float32)],
    )(xt, row_embed, col_embed)
    return out.reshape(b, h, w, c).transpose(0, 3, 1, 2)


# speedup vs baseline: 3.2007x; 1.0766x over previous
import functools

import jax
import jax.numpy as jnp
from jax.experimental import pallas as pl
from jax.experimental.pallas import tpu as pltpu


def _posenc_kernel(x_ref, row_ref, col_ref, o_ref, pos_ref, *, H, W, CH):
    i = pl.program_id(0)
    HW = H * W

    @pl.when(i == 0)
    def _build_pos():
        r = jax.lax.broadcasted_iota(jnp.int32, (HW, H), 0)
        k = jax.lax.broadcasted_iota(jnp.int32, (HW, H), 1)
        Eh = (r // W == k).astype(jnp.float32)
        Ew = (r % W == k).astype(jnp.float32)
        pos_ref[:, :CH] = jax.lax.dot(
            Eh, row_ref[:H, :], precision=jax.lax.Precision.HIGHEST,
            preferred_element_type=jnp.float32)
        pos_ref[:, CH:] = jax.lax.dot(
            Ew, col_ref[:W, :], precision=jax.lax.Precision.HIGHEST,
            preferred_element_type=jnp.float32)

    o_ref[...] = x_ref[...] + pos_ref[...][None]


def kernel(x, row_embed, col_embed):
    b, c, h, w = x.shape
    ch = c // 2
    hw = h * w
    G = 8
    xt = jnp.transpose(x, (0, 2, 3, 1)).reshape(b, hw, c)
    body = functools.partial(_posenc_kernel, H=h, W=w, CH=ch)
    out = pl.pallas_call(
        body,
        grid=(b // G,),
        in_specs=[
            pl.BlockSpec((G, hw, c), lambda i: (i, 0, 0)),
            pl.BlockSpec(row_embed.shape, lambda i: (0, 0)),
            pl.BlockSpec(col_embed.shape, lambda i: (0, 0)),
        ],
        out_specs=pl.BlockSpec((G, hw, c), lambda i: (i, 0, 0)),
        out_shape=jax.ShapeDtypeStruct((b, hw, c), x.dtype),
        scratch_shapes=[pltpu.VMEM((hw, c), jnp.float32)],
    )(xt, row_embed, col_embed)
    return out.reshape(b, h, w, c).transpose(0, 3, 1, 2)


# speedup vs baseline: 3.4910x; 1.0907x over previous
import functools

import jax
import jax.numpy as jnp
from jax.experimental import pallas as pl
from jax.experimental.pallas import tpu as pltpu


def _posenc_kernel(x_ref, row_ref, col_ref, o_ref, pos_ref, *, H, W, CH):
    i = pl.program_id(0)
    HW = H * W

    @pl.when(i == 0)
    def _build_pos():
        r = jax.lax.broadcasted_iota(jnp.int32, (HW, H), 0)
        k = jax.lax.broadcasted_iota(jnp.int32, (HW, H), 1)
        Eh = (r // W == k).astype(jnp.float32)
        Ew = (r % W == k).astype(jnp.float32)
        pos_ref[:, :CH] = jax.lax.dot(
            Eh, row_ref[:H, :], precision=jax.lax.Precision.HIGHEST,
            preferred_element_type=jnp.float32)
        pos_ref[:, CH:] = jax.lax.dot(
            Ew, col_ref[:W, :], precision=jax.lax.Precision.HIGHEST,
            preferred_element_type=jnp.float32)

    o_ref[...] = x_ref[...] + pos_ref[...][None]


def kernel(x, row_embed, col_embed):
    b, c, h, w = x.shape
    ch = c // 2
    hw = h * w
    G = 16
    xt = jnp.transpose(x, (0, 2, 3, 1)).reshape(b, hw, c)
    body = functools.partial(_posenc_kernel, H=h, W=w, CH=ch)
    out = pl.pallas_call(
        body,
        grid=(b // G,),
        in_specs=[
            pl.BlockSpec((G, hw, c), lambda i: (i, 0, 0)),
            pl.BlockSpec(row_embed.shape, lambda i: (0, 0)),
            pl.BlockSpec(col_embed.shape, lambda i: (0, 0)),
        ],
        out_specs=pl.BlockSpec((G, hw, c), lambda i: (i, 0, 0)),
        out_shape=jax.ShapeDtypeStruct((b, hw, c), x.dtype),
        scratch_shapes=[pltpu.VMEM((hw, c), jnp.float32)],
    )(xt, row_embed, col_embed)
    return out.reshape(b, h, w, c).transpose(0, 3, 1, 2)
